# baseline (speedup 1.0000x reference)
import functools

import jax
import jax.numpy as jnp
from jax import lax
from jax.experimental import pallas as pl
from jax.experimental.pallas import tpu as pltpu
from jax.experimental.pallas import tpu_sc as plsc

_L = 16


@functools.lru_cache(maxsize=None)
def _build_warp(N, C, H, W):
    HW = H * W
    P = N * HW
    info = plsc.get_sparse_core_info()
    NC, NS = info.num_cores, info.num_subcores
    NW = NC * NS
    PPW = P // NW
    assert P % NW == 0 and HW % PPW == 0
    B = 96
    NCHUNK = PPW // B
    assert PPW % B == 0 and NCHUNK % 2 == 0
    KG = C // _L
    assert C % _L == 0

    mesh = plsc.VectorSubcoreMesh(core_axis_name="c", subcore_axis_name="s")

    @functools.partial(
        pl.kernel,
        out_type=jax.ShapeDtypeStruct((N, HW, C), jnp.float32),
        mesh=mesh,
        compiler_params=pltpu.CompilerParams(use_tc_tiling_on_sc=False),
        scratch_types=[
            pltpu.VMEM((2, B), jnp.float32),
            pltpu.VMEM((2, B), jnp.float32),
            pltpu.VMEM((2, B), jnp.int32),
            pltpu.VMEM((2, B), jnp.int32),
            pltpu.VMEM((2, B), jnp.int32),
            pltpu.VMEM((2, B), jnp.int32),
            pltpu.VMEM((2, B), jnp.float32),
            pltpu.VMEM((2, B), jnp.float32),
            pltpu.VMEM((2, B), jnp.float32),
            pltpu.VMEM((2, B), jnp.float32),
            pltpu.VMEM((2, B, C), jnp.float32),
            pltpu.VMEM((2, B, C), jnp.float32),
            pltpu.VMEM((2, B, C), jnp.float32),
            pltpu.VMEM((2, B, C), jnp.float32),
            pltpu.VMEM((2, B, C), jnp.float32),
            pltpu.SemaphoreType.DMA,
            pltpu.SemaphoreType.DMA,
            pltpu.SemaphoreType.DMA,
            pltpu.SemaphoreType.DMA,
            pltpu.SemaphoreType.DMA,
            pltpu.SemaphoreType.DMA,
        ],
    )
    def warp(table_hbm, fy_hbm, fx_hbm, out_hbm,
             fy_v, fx_v, i0, i1, i2, i3, w0, w1, w2, w3,
             b0, b1, b2, b3, ob,
             semf0, semf1, semg0, semg1, semo0, semo1):
        wid = lax.axis_index("s") * NC + lax.axis_index("c")
        wpi = HW // PPW
        nb = wid // wpi
        base = (wid - nb * wpi) * PPW
        semf = (semf0, semf1)
        semg = (semg0, semg1)
        semo = (semo0, semo1)
        idx_refs = (i0, i1, i2, i3)
        buf_refs = (b0, b1, b2, b3)

        def fire_flow(chunk, s):
            p0 = base + chunk * B
            pltpu.async_copy(fy_hbm.at[nb, pl.ds(p0, B)], fy_v.at[s], semf[s])
            pltpu.async_copy(fx_hbm.at[nb, pl.ds(p0, B)], fx_v.at[s], semf[s])

        def wait_flow(s):
            pltpu.make_async_copy(fy_hbm.at[0, pl.ds(0, B)], fy_v.at[s], semf[s]).wait()
            pltpu.make_async_copy(fx_hbm.at[0, pl.ds(0, B)], fx_v.at[s], semf[s]).wait()

        def compute_idx(chunk, s):
            p0 = base + chunk * B

            def idx_body(j, _):
                sl = pl.ds(j * _L, _L)
                fullf = lambda v: jnp.full((_L,), v, jnp.float32)
                fulli = lambda v: jnp.full((_L,), v, jnp.int32)
                p = fulli(p0 + j * _L) + lax.iota(jnp.int32, 16)
                hh = lax.div(p, fulli(W))
                ww = p - hh * fulli(W)
                iy = (fullf(2.0) * ((hh.astype(jnp.float32) + fy_v[s, sl]) / fullf(H - 1) - fullf(0.5)) + fullf(1.0)) / fullf(2.0) * fullf(H - 1)
                ix = (fullf(2.0) * ((ww.astype(jnp.float32) + fx_v[s, sl]) / fullf(W - 1) - fullf(0.5)) + fullf(1.0)) / fullf(2.0) * fullf(W - 1)
                iy = jnp.minimum(jnp.maximum(iy, fullf(-8.0)), fullf(H + 8.0))
                ix = jnp.minimum(jnp.maximum(ix, fullf(-8.0)), fullf(W + 8.0))
                ty = iy.astype(jnp.int32)
                tx = ix.astype(jnp.int32)
                yf = ty - jnp.where(ty.astype(jnp.float32) > iy, fulli(1), fulli(0))
                xf = tx - jnp.where(tx.astype(jnp.float32) > ix, fulli(1), fulli(0))
                yff = yf.astype(jnp.float32)
                xff = xf.astype(jnp.float32)
                fy1 = (yff + fullf(1.0)) - iy
                fy0 = iy - yff
                fx1 = (xff + fullf(1.0)) - ix
                fx0 = ix - xff
                y0 = jnp.minimum(jnp.maximum(yf, fulli(0)), fulli(H - 1))
                y1 = jnp.minimum(jnp.maximum(yf + fulli(1), fulli(0)), fulli(H - 1))
                x0 = jnp.minimum(jnp.maximum(xf, fulli(0)), fulli(W - 1))
                x1 = jnp.minimum(jnp.maximum(xf + fulli(1), fulli(0)), fulli(W - 1))
                rn = y0 * fulli(W)
                rs = y1 * fulli(W)
                i0[s, sl] = rn + x0
                i1[s, sl] = rn + x1
                i2[s, sl] = rs + x0
                i3[s, sl] = rs + x1
                w0[s, sl] = fx1 * fy1
                w1[s, sl] = fx0 * fy1
                w2[s, sl] = fx1 * fy0
                w3[s, sl] = fx0 * fy0
                return 0

            lax.fori_loop(0, B // _L, idx_body, 0)

        def fire_gathers(s):
            for k in range(4):
                pltpu.async_copy(table_hbm.at[nb].at[idx_refs[k].at[s]],
                                 buf_refs[k].at[s], semg[s])

        def wait_gathers(s):
            for k in range(4):
                pltpu.make_async_copy(table_hbm.at[nb].at[idx_refs[k].at[s]],
                                      buf_refs[k].at[s], semg[s]).wait()

        def combine(s):
            def grp_body(g, _):
                sw = pl.ds(g * _L, _L)
                wv0, wv1, wv2, wv3 = w0[s, sw], w1[s, sw], w2[s, sw], w3[s, sw]
                for e in range(_L):
                    i = g * _L + e
                    v0 = jnp.broadcast_to(wv0[e], (_L,))
                    v1 = jnp.broadcast_to(wv1[e], (_L,))
                    v2 = jnp.broadcast_to(wv2[e], (_L,))
                    v3 = jnp.broadcast_to(wv3[e], (_L,))
                    for k in range(KG):
                        sl = pl.ds(k * _L, _L)
                        ob[s, i, sl] = (b0[s, i, sl] * v0 + b1[s, i, sl] * v1
                                        + b2[s, i, sl] * v2 + b3[s, i, sl] * v3)
                return 0

            lax.fori_loop(0, B // _L, grp_body, 0)

        def fire_store(chunk, s):
            p0 = base + chunk * B
            pltpu.async_copy(ob.at[s], out_hbm.at[nb, pl.ds(p0, B)], semo[s])

        def drain_store(s):
            pltpu.make_async_copy(ob.at[s], out_hbm.at[0, pl.ds(0, B)], semo[s]).wait()

        fire_flow(0, 0)
        fire_flow(1, 1)
        wait_flow(0)
        compute_idx(0, 0)
        fire_gathers(0)

        def body(c2, _):
            c = c2 * 2
            @pl.when(c2 < NCHUNK // 2 - 1)
            def _():
                fire_flow(c + 2, 0)
            wait_flow(1)
            compute_idx(c + 1, 1)
            fire_gathers(1)
            wait_gathers(0)

            @pl.when(c2 >= 1)
            def _():
                drain_store(0)
            combine(0)
            fire_store(c, 0)

            @pl.when(c2 < NCHUNK // 2 - 1)
            def _():
                fire_flow(c + 3, 1)
                wait_flow(0)
                compute_idx(c + 2, 0)
                fire_gathers(0)
            wait_gathers(1)

            @pl.when(c2 >= 1)
            def _():
                drain_store(1)
            combine(1)
            fire_store(c + 1, 1)
            return 0

        lax.fori_loop(0, NCHUNK // 2, body, 0)
        drain_store(0)
        drain_store(1)

    return warp


def kernel(src, flow):
    N, C, H, W = src.shape
    table = src.reshape(N, C, H * W).transpose(0, 2, 1)
    fy = flow[:, 0].reshape(N, H * W)
    fx = flow[:, 1].reshape(N, H * W)
    out_t = _build_warp(N, C, H, W)(table, fy, fx)
    return out_t.transpose(0, 2, 1).reshape(N, C, H, W)


# speedup vs baseline: 1.1617x; 1.1617x over previous
import functools

import jax
import jax.numpy as jnp
from jax import lax
from jax.experimental import pallas as pl
from jax.experimental.pallas import tpu as pltpu
from jax.experimental.pallas import tpu_sc as plsc

_L = 16


@functools.lru_cache(maxsize=None)
def _build_warp(N, C, H, W):
    HW = H * W
    P = N * HW
    info = plsc.get_sparse_core_info()
    NC, NS = info.num_cores, info.num_subcores
    NW = NC * NS
    PPW = HW // NW
    assert HW % NW == 0
    B = 96
    NCHUNK = PPW // B
    assert PPW % B == 0 and NCHUNK % 2 == 0
    KG = C // _L
    assert C % _L == 0

    mesh = plsc.VectorSubcoreMesh(core_axis_name="c", subcore_axis_name="s")

    @functools.partial(
        pl.kernel,
        out_type=jax.ShapeDtypeStruct((HW, C), jnp.float32),
        mesh=mesh,
        compiler_params=pltpu.CompilerParams(use_tc_tiling_on_sc=False),
        scratch_types=[
            pltpu.VMEM((2, B), jnp.float32),
            pltpu.VMEM((2, B), jnp.float32),
            pltpu.VMEM((2, B), jnp.int32),
            pltpu.VMEM((2, B), jnp.int32),
            pltpu.VMEM((2, B), jnp.int32),
            pltpu.VMEM((2, B), jnp.int32),
            pltpu.VMEM((2, B), jnp.float32),
            pltpu.VMEM((2, B), jnp.float32),
            pltpu.VMEM((2, B), jnp.float32),
            pltpu.VMEM((2, B), jnp.float32),
            pltpu.VMEM((2, B, C), jnp.float32),
            pltpu.VMEM((2, B, C), jnp.float32),
            pltpu.VMEM((2, B, C), jnp.float32),
            pltpu.VMEM((2, B, C), jnp.float32),
            pltpu.VMEM((2, B, C), jnp.float32),
            pltpu.SemaphoreType.DMA,
            pltpu.SemaphoreType.DMA,
            pltpu.SemaphoreType.DMA,
            pltpu.SemaphoreType.DMA,
            pltpu.SemaphoreType.DMA,
            pltpu.SemaphoreType.DMA,
        ],
    )
    def warp(table_hbm, fy_hbm, fx_hbm, out_hbm,
             fy_v, fx_v, i0, i1, i2, i3, w0, w1, w2, w3,
             b0, b1, b2, b3, ob,
             semf0, semf1, semg0, semg1, semo0, semo1):
        wid = lax.axis_index("s") * NC + lax.axis_index("c")
        base = wid * PPW
        semf = (semf0, semf1)
        semg = (semg0, semg1)
        semo = (semo0, semo1)
        idx_refs = (i0, i1, i2, i3)
        buf_refs = (b0, b1, b2, b3)

        def fire_flow(chunk, s):
            p0 = base + chunk * B
            pltpu.async_copy(fy_hbm.at[pl.ds(p0, B)], fy_v.at[s], semf[s])
            pltpu.async_copy(fx_hbm.at[pl.ds(p0, B)], fx_v.at[s], semf[s])

        def wait_flow(s):
            pltpu.make_async_copy(fy_hbm.at[pl.ds(0, B)], fy_v.at[s], semf[s]).wait()
            pltpu.make_async_copy(fx_hbm.at[pl.ds(0, B)], fx_v.at[s], semf[s]).wait()

        def compute_idx(chunk, s):
            p0 = base + chunk * B

            def idx_body(j, _):
                sl = pl.ds(j * _L, _L)
                fullf = lambda v: jnp.full((_L,), v, jnp.float32)
                fulli = lambda v: jnp.full((_L,), v, jnp.int32)
                p = fulli(p0 + j * _L) + lax.iota(jnp.int32, 16)
                hh = lax.div(p, fulli(W))
                ww = p - hh * fulli(W)
                iy = (fullf(2.0) * ((hh.astype(jnp.float32) + fy_v[s, sl]) / fullf(H - 1) - fullf(0.5)) + fullf(1.0)) / fullf(2.0) * fullf(H - 1)
                ix = (fullf(2.0) * ((ww.astype(jnp.float32) + fx_v[s, sl]) / fullf(W - 1) - fullf(0.5)) + fullf(1.0)) / fullf(2.0) * fullf(W - 1)
                iy = jnp.minimum(jnp.maximum(iy, fullf(-8.0)), fullf(H + 8.0))
                ix = jnp.minimum(jnp.maximum(ix, fullf(-8.0)), fullf(W + 8.0))
                ty = iy.astype(jnp.int32)
                tx = ix.astype(jnp.int32)
                yf = ty - jnp.where(ty.astype(jnp.float32) > iy, fulli(1), fulli(0))
                xf = tx - jnp.where(tx.astype(jnp.float32) > ix, fulli(1), fulli(0))
                yff = yf.astype(jnp.float32)
                xff = xf.astype(jnp.float32)
                fy1 = (yff + fullf(1.0)) - iy
                fy0 = iy - yff
                fx1 = (xff + fullf(1.0)) - ix
                fx0 = ix - xff
                y0 = jnp.minimum(jnp.maximum(yf, fulli(0)), fulli(H - 1))
                y1 = jnp.minimum(jnp.maximum(yf + fulli(1), fulli(0)), fulli(H - 1))
                x0 = jnp.minimum(jnp.maximum(xf, fulli(0)), fulli(W - 1))
                x1 = jnp.minimum(jnp.maximum(xf + fulli(1), fulli(0)), fulli(W - 1))
                rn = y0 * fulli(W)
                rs = y1 * fulli(W)
                i0[s, sl] = rn + x0
                i1[s, sl] = rn + x1
                i2[s, sl] = rs + x0
                i3[s, sl] = rs + x1
                w0[s, sl] = fx1 * fy1
                w1[s, sl] = fx0 * fy1
                w2[s, sl] = fx1 * fy0
                w3[s, sl] = fx0 * fy0
                return 0

            lax.fori_loop(0, B // _L, idx_body, 0)

        def fire_gathers(s):
            for k in range(4):
                pltpu.async_copy(table_hbm.at[idx_refs[k].at[s]],
                                 buf_refs[k].at[s], semg[s])

        def wait_gathers(s):
            for k in range(4):
                pltpu.make_async_copy(table_hbm.at[idx_refs[k].at[s]],
                                      buf_refs[k].at[s], semg[s]).wait()

        def combine(s):
            def grp_body(g, _):
                sw = pl.ds(g * _L, _L)
                wv0, wv1, wv2, wv3 = w0[s, sw], w1[s, sw], w2[s, sw], w3[s, sw]
                for e in range(_L):
                    i = g * _L + e
                    v0 = jnp.broadcast_to(wv0[e], (_L,))
                    v1 = jnp.broadcast_to(wv1[e], (_L,))
                    v2 = jnp.broadcast_to(wv2[e], (_L,))
                    v3 = jnp.broadcast_to(wv3[e], (_L,))
                    for k in range(KG):
                        sl = pl.ds(k * _L, _L)
                        ob[s, i, sl] = (b0[s, i, sl] * v0 + b1[s, i, sl] * v1
                                        + b2[s, i, sl] * v2 + b3[s, i, sl] * v3)
                return 0

            lax.fori_loop(0, B // _L, grp_body, 0)

        def fire_store(chunk, s):
            p0 = base + chunk * B
            pltpu.async_copy(ob.at[s], out_hbm.at[pl.ds(p0, B)], semo[s])

        def drain_store(s):
            pltpu.make_async_copy(ob.at[s], out_hbm.at[pl.ds(0, B)], semo[s]).wait()

        fire_flow(0, 0)
        fire_flow(1, 1)
        wait_flow(0)
        compute_idx(0, 0)
        fire_gathers(0)

        def body(c2, _):
            c = c2 * 2
            @pl.when(c2 < NCHUNK // 2 - 1)
            def _():
                fire_flow(c + 2, 0)
            wait_flow(1)
            compute_idx(c + 1, 1)
            fire_gathers(1)
            wait_gathers(0)

            @pl.when(c2 >= 1)
            def _():
                drain_store(0)
            combine(0)
            fire_store(c, 0)

            @pl.when(c2 < NCHUNK // 2 - 1)
            def _():
                fire_flow(c + 3, 1)
                wait_flow(0)
                compute_idx(c + 2, 0)
                fire_gathers(0)
            wait_gathers(1)

            @pl.when(c2 >= 1)
            def _():
                drain_store(1)
            combine(1)
            fire_store(c + 1, 1)
            return 0

        lax.fori_loop(0, NCHUNK // 2, body, 0)
        drain_store(0)
        drain_store(1)

    return warp


def kernel(src, flow):
    N, C, H, W = src.shape
    warp = _build_warp(N, C, H, W)
    outs = []
    for n in range(N):
        table_n = src[n].reshape(C, H * W).transpose(1, 0)
        fy_n = flow[n, 0].reshape(H * W)
        fx_n = flow[n, 1].reshape(H * W)
        outs.append(warp(table_n, fy_n, fx_n).transpose(1, 0))
    return jnp.stack(outs).reshape(N, C, H, W)
